# baseline (speedup 1.0000x reference)
import jax
import jax.numpy as jnp
from jax import lax
from jax.experimental import pallas as pl
from jax.experimental.pallas import tpu as pltpu

NBUF = 8
CHUNK = 32


def _body(top_smem, top_ref, shifted_ref, trees_hbm, hid_hbm,
          out_hbm, head_hbm, inbuf, outbuf, gbuf, in_sems, out_sems, gsem):
    m, s, i = trees_hbm.shape

    n_chunks = m // CHUNK

    def in_copy(c, b):
        return pltpu.make_async_copy(
            trees_hbm.at[pl.ds(c * CHUNK, CHUNK)],
            inbuf.at[pl.ds(b * CHUNK, CHUNK)],
            in_sems.at[b],
        )

    def out_copy(c, b):
        return pltpu.make_async_copy(
            outbuf.at[pl.ds(b * CHUNK, CHUNK)],
            out_hbm.at[pl.ds(c * CHUNK, CHUNK)],
            out_sems.at[b],
        )

    for b in range(NBUF):
        in_copy(b, b).start()

    def step(c, carry):
        b = lax.rem(c, NBUF)

        for k in range(CHUNK):
            j = c * CHUNK + k
            t = top_smem[j]
            pltpu.make_async_copy(
                hid_hbm.at[j, pl.ds(t + 1, 1)],
                gbuf.at[pl.ds(j, 1)],
                gsem,
            ).start()

        @pl.when(c >= NBUF)
        def _():
            out_copy(c - NBUF, b).wait()

        in_copy(c, b).wait()
        rows = inbuf[pl.ds(b * CHUNK, CHUNK)]
        top = top_ref[pl.ds(c * CHUNK, CHUNK)]
        shifted = shifted_ref[pl.ds(c * CHUNK, CHUNK)]
        stack_iota = lax.broadcasted_iota(jnp.int32, (CHUNK, s, i), 1)
        outbuf[pl.ds(b * CHUNK, CHUNK)] = jnp.where(
            stack_iota == top, shifted, rows
        )
        out_copy(c, b).start()

        @pl.when(c + NBUF < n_chunks)
        def _():
            in_copy(c + NBUF, b).start()

        return carry

    lax.fori_loop(0, n_chunks, step, 0)
    for b in range(NBUF):
        c = n_chunks - NBUF + b
        out_copy(c, c % NBUF).wait()
    pltpu.make_async_copy(head_hbm, gbuf, gsem).wait()
    pltpu.sync_copy(gbuf, head_hbm)


def kernel(trees, hiddens, shifted_embs, top_position):
    m, s, i = trees.shape
    h = hiddens.shape[2]
    call = pl.pallas_call(
        _body,
        in_specs=[
            pl.BlockSpec(memory_space=pltpu.SMEM),
            pl.BlockSpec(memory_space=pltpu.VMEM),
            pl.BlockSpec(memory_space=pltpu.VMEM),
            pl.BlockSpec(memory_space=pltpu.HBM),
            pl.BlockSpec(memory_space=pltpu.HBM),
        ],
        out_specs=(
            pl.BlockSpec(memory_space=pltpu.HBM),
            pl.BlockSpec(memory_space=pltpu.HBM),
        ),
        scratch_shapes=[
            pltpu.VMEM((NBUF * CHUNK, s, i), trees.dtype),
            pltpu.VMEM((NBUF * CHUNK, s, i), trees.dtype),
            pltpu.VMEM((m, h), hiddens.dtype),
            pltpu.SemaphoreType.DMA((NBUF,)),
            pltpu.SemaphoreType.DMA((NBUF,)),
            pltpu.SemaphoreType.DMA,
        ],
        out_shape=(
            jax.ShapeDtypeStruct((m, s, i), trees.dtype),
            jax.ShapeDtypeStruct((m, h), hiddens.dtype),
        ),
    )
    new_trees, hidden_head = call(
        top_position,
        top_position.reshape(m, 1, 1),
        shifted_embs.reshape(m, 1, i),
        trees,
        hiddens,
    )
    return (new_trees, hidden_head)
